# baseline (speedup 1.0000x reference)
import functools

import jax
import jax.numpy as jnp
from jax import lax
from jax.experimental import pallas as pl
from jax.experimental.pallas import tpu as pltpu
from jax.experimental.pallas import tpu_sc as plsc

B = 16384
V = 100000
H = 128
D = 64

_CHUNKS = (8192, 4096, 4096)


_NC = 2
_NS = 16
_NW = _NC * _NS
_CHUNK = 128
_NBUF = 4


def _gather_body(nch, x0_hbm, x1_hbm, e0_hbm, e1_hbm, out0, out1,
                 idx0_v, idx1_v, r0, r1, r2, r3,
                 g0, g1, g2, g3, t0, t1, t2, t3):
    bpw = nch * _CHUNK
    wid = lax.axis_index("s") * _NC + lax.axis_index("c")
    rows = (r0, r1, r2, r3)
    gsems = (g0, g1, g2, g3)
    ssems = (t0, t1, t2, t3)
    base = wid * bpw
    pltpu.sync_copy(x0_hbm.at[pl.ds(wid * nch, nch)], idx0_v)
    pltpu.sync_copy(x1_hbm.at[pl.ds(wid * nch, nch)], idx1_v)
    tasks = [(e0_hbm, idx0_v, out0, j) for j in range(nch)] + \
            [(e1_hbm, idx1_v, out1, j) for j in range(nch)]
    nt = len(tasks)
    nb = min(_NBUF, nt)
    gh = [None] * nb
    sh = [None] * nt
    for t in range(nb):
        table, idx_v, _, j = tasks[t]
        gh[t] = pltpu.async_copy(table.at[idx_v.at[j]], rows[t], gsems[t])
    for t in range(nt):
        b = t % nb
        gh[b].wait()
        _, _, out, j = tasks[t]
        sh[t] = pltpu.async_copy(rows[b], out.at[pl.ds(base + j * _CHUNK,
                                                       _CHUNK)], ssems[b])
        if t + nb < nt:
            sh[t].wait()
            table, idx_v, _, j2 = tasks[t + nb]
            gh[b] = pltpu.async_copy(table.at[idx_v.at[j2]], rows[b],
                                     gsems[b])
    for t in range(max(0, nt - nb), nt):
        sh[t].wait()


@functools.cache
def _make_sc_gather(nr):
    nch = nr // _NW // _CHUNK
    return pl.kernel(
        functools.partial(_gather_body, nch),
        mesh=plsc.VectorSubcoreMesh(core_axis_name="c", subcore_axis_name="s"),
        out_type=[
            jax.ShapeDtypeStruct((nr, H), jnp.float32),
            jax.ShapeDtypeStruct((nr, H), jnp.float32),
        ],
        scratch_types=[
            pltpu.VMEM((nch, _CHUNK), jnp.int32),
            pltpu.VMEM((nch, _CHUNK), jnp.int32),
            pltpu.VMEM((_CHUNK, H), jnp.float32),
            pltpu.VMEM((_CHUNK, H), jnp.float32),
            pltpu.VMEM((_CHUNK, H), jnp.float32),
            pltpu.VMEM((_CHUNK, H), jnp.float32),
            pltpu.SemaphoreType.DMA,
            pltpu.SemaphoreType.DMA,
            pltpu.SemaphoreType.DMA,
            pltpu.SemaphoreType.DMA,
            pltpu.SemaphoreType.DMA,
            pltpu.SemaphoreType.DMA,
            pltpu.SemaphoreType.DMA,
            pltpu.SemaphoreType.DMA,
        ],
    )


_BS = 4096


def _silu(v):
    return v * jax.nn.sigmoid(v)


def _mlp_math(e0, e1, w1a, w1b, w2, w3, mavg, b1, b2, b3):
    z = jnp.dot(e0, w1a, preferred_element_type=jnp.float32)
    z += jnp.dot(e1, w1b, preferred_element_type=jnp.float32)
    z = _silu(z + b1)
    c = z - jnp.dot(z, mavg, preferred_element_type=jnp.float32)
    var = jnp.dot(c * c, mavg, preferred_element_type=jnp.float32)
    z = c * lax.rsqrt(var + 1e-5)
    z = _silu(jnp.dot(z, w2, preferred_element_type=jnp.float32) + b2)
    return jnp.dot(z, w3, preferred_element_type=jnp.float32) + b3


def _mlp_body(e0_ref, e1_ref, w1a_ref, w1b_ref, w2_ref, w3_ref, mavg_ref,
              b1_ref, b2_ref, b3_ref, out_ref):
    out_ref[...] = _mlp_math(
        e0_ref[...], e1_ref[...], w1a_ref[...], w1b_ref[...], w2_ref[...],
        w3_ref[...], mavg_ref[...], b1_ref[...], b2_ref[...], b3_ref[...])


def _mlp_body_aliased(prev_ref, e0_ref, e1_ref, w1a_ref, w1b_ref, w2_ref,
                      w3_ref, mavg_ref, b1_ref, b2_ref, b3_ref, out_ref):
    del prev_ref
    out_ref[...] = _mlp_math(
        e0_ref[...], e1_ref[...], w1a_ref[...], w1b_ref[...], w2_ref[...],
        w3_ref[...], mavg_ref[...], b1_ref[...], b2_ref[...], b3_ref[...])


def _mlp_chunk(row_off, prev, e0c, e1c, weights):
    bs = min(_BS, e0c.shape[0])
    nblk = e0c.shape[0] // bs
    off = row_off // bs
    row_spec = pl.BlockSpec((bs, H), lambda i: (i, 0))
    w_spec = pl.BlockSpec((H, H), lambda i: (0, 0))
    b_spec = pl.BlockSpec((1, H), lambda i: (0, 0))
    out_spec = pl.BlockSpec((bs, H), lambda i: (i + off, 0))
    common = dict(
        grid=(nblk,),
        out_specs=out_spec,
        out_shape=jax.ShapeDtypeStruct((B, H), jnp.float32),
    )
    wspecs = [w_spec] * 5 + [b_spec] * 3
    if prev is None:
        return pl.pallas_call(
            _mlp_body,
            in_specs=[row_spec, row_spec] + wspecs,
            **common,
        )(e0c, e1c, *weights)
    return pl.pallas_call(
        _mlp_body_aliased,
        in_specs=[pl.BlockSpec(memory_space=pl.ANY), row_spec, row_spec]
                 + wspecs,
        input_output_aliases={0: 0},
        **common,
    )(prev, e0c, e1c, *weights)


def kernel(x, E0, E1,
           W1_0, b1_0, W2_0, b2_0, W3_0, b3_0,
           W1_1, b1_1, W2_1, b2_1, W3_1, b3_1):
    w1 = jnp.concatenate([W1_0.T, W1_1.T], axis=1)
    w1a, w1b = w1[:H], w1[H:]
    zblk = jnp.zeros((D, D), jnp.float32)
    w2bd = jnp.block([[W2_0.T, zblk], [zblk, W2_1.T]])
    w3bd = jnp.block([[W3_0.T, zblk], [zblk, W3_1.T]])
    ones = jnp.full((D, D), 1.0 / D, jnp.float32)
    mavg = jnp.block([[ones, zblk], [zblk, ones]])
    b1 = jnp.concatenate([b1_0, b1_1]).reshape(1, 2 * D)
    b2 = jnp.concatenate([b2_0, b2_1]).reshape(1, 2 * D)
    b3 = jnp.concatenate([b3_0, b3_1]).reshape(1, 2 * D)
    weights = (w1a, w1b, w2bd, w3bd, mavg, b1, b2, b3)

    x0 = x[:, 0].reshape(B // _CHUNK, _CHUNK)
    x1 = x[:, 1].reshape(B // _CHUNK, _CHUNK)
    pairs = []
    off = 0
    for nr in _CHUNKS:
        r0, r1 = off // _CHUNK, (off + nr) // _CHUNK
        pairs.append((off, _make_sc_gather(nr)(x0[r0:r1], x1[r0:r1], E0, E1)))
        off += nr
    out = None
    for off, (e0c, e1c) in pairs:
        out = _mlp_chunk(off, out, e0c, e1c, weights)
    return out


# speedup vs baseline: 1.0994x; 1.0994x over previous
import functools

import jax
import jax.numpy as jnp
from jax import lax
from jax.experimental import pallas as pl
from jax.experimental.pallas import tpu as pltpu
from jax.experimental.pallas import tpu_sc as plsc

B = 16384
V = 100000
H = 128
D = 64

_CHUNKS = (8192, 8192)


_NC = 2
_NS = 16
_NW = _NC * _NS
_CHUNK = 128
_NBUF = 4


def _gather_body(nch, x0_hbm, x1_hbm, e0_hbm, e1_hbm, out0, out1,
                 idx0_v, idx1_v, r0, r1, r2, r3,
                 g0, g1, g2, g3, t0, t1, t2, t3):
    bpw = nch * _CHUNK
    wid = lax.axis_index("s") * _NC + lax.axis_index("c")
    rows = (r0, r1, r2, r3)
    gsems = (g0, g1, g2, g3)
    ssems = (t0, t1, t2, t3)
    base = wid * bpw
    tasks = [(e0_hbm, idx0_v, out0, j) for j in range(nch)] + \
            [(e1_hbm, idx1_v, out1, j) for j in range(nch)]
    nt = len(tasks)
    nb = min(_NBUF, nt)
    gh = [None] * nb
    sh = [None] * nt
    pltpu.sync_copy(x0_hbm.at[pl.ds(wid * nch, nch)], idx0_v)
    for t in range(min(nch, nb)):
        table, idx_v, _, j = tasks[t]
        gh[t] = pltpu.async_copy(table.at[idx_v.at[j]], rows[t], gsems[t])
    pltpu.sync_copy(x1_hbm.at[pl.ds(wid * nch, nch)], idx1_v)
    for t in range(min(nch, nb), nb):
        table, idx_v, _, j = tasks[t]
        gh[t] = pltpu.async_copy(table.at[idx_v.at[j]], rows[t], gsems[t])
    for t in range(nt):
        b = t % nb
        gh[b].wait()
        _, _, out, j = tasks[t]
        sh[t] = pltpu.async_copy(rows[b], out.at[pl.ds(base + j * _CHUNK,
                                                       _CHUNK)], ssems[b])
        if t + nb < nt:
            sh[t].wait()
            table, idx_v, _, j2 = tasks[t + nb]
            gh[b] = pltpu.async_copy(table.at[idx_v.at[j2]], rows[b],
                                     gsems[b])
    for t in range(max(0, nt - nb), nt):
        sh[t].wait()


@functools.cache
def _make_sc_gather(nr):
    nch = nr // _NW // _CHUNK
    return pl.kernel(
        functools.partial(_gather_body, nch),
        mesh=plsc.VectorSubcoreMesh(core_axis_name="c", subcore_axis_name="s"),
        out_type=[
            jax.ShapeDtypeStruct((nr, H), jnp.float32),
            jax.ShapeDtypeStruct((nr, H), jnp.float32),
        ],
        scratch_types=[
            pltpu.VMEM((nch, _CHUNK), jnp.int32),
            pltpu.VMEM((nch, _CHUNK), jnp.int32),
            pltpu.VMEM((_CHUNK, H), jnp.float32),
            pltpu.VMEM((_CHUNK, H), jnp.float32),
            pltpu.VMEM((_CHUNK, H), jnp.float32),
            pltpu.VMEM((_CHUNK, H), jnp.float32),
            pltpu.SemaphoreType.DMA,
            pltpu.SemaphoreType.DMA,
            pltpu.SemaphoreType.DMA,
            pltpu.SemaphoreType.DMA,
            pltpu.SemaphoreType.DMA,
            pltpu.SemaphoreType.DMA,
            pltpu.SemaphoreType.DMA,
            pltpu.SemaphoreType.DMA,
        ],
    )


_BS = 4096


def _silu(v):
    return v * jax.nn.sigmoid(v)


def _mlp_math(e0, e1, w1a, w1b, w2, w3, mavg, b1, b2, b3):
    z = jnp.dot(e0, w1a, preferred_element_type=jnp.float32)
    z += jnp.dot(e1, w1b, preferred_element_type=jnp.float32)
    z = _silu(z + b1)
    c = z - jnp.dot(z, mavg, preferred_element_type=jnp.float32)
    var = jnp.dot(c * c, mavg, preferred_element_type=jnp.float32)
    z = c * lax.rsqrt(var + 1e-5)
    z = _silu(jnp.dot(z, w2, preferred_element_type=jnp.float32) + b2)
    return jnp.dot(z, w3, preferred_element_type=jnp.float32) + b3


def _mlp_body(e0_ref, e1_ref, w1a_ref, w1b_ref, w2_ref, w3_ref, mavg_ref,
              b1_ref, b2_ref, b3_ref, out_ref):
    out_ref[...] = _mlp_math(
        e0_ref[...], e1_ref[...], w1a_ref[...], w1b_ref[...], w2_ref[...],
        w3_ref[...], mavg_ref[...], b1_ref[...], b2_ref[...], b3_ref[...])


def _mlp_body_aliased(prev_ref, e0_ref, e1_ref, w1a_ref, w1b_ref, w2_ref,
                      w3_ref, mavg_ref, b1_ref, b2_ref, b3_ref, out_ref):
    del prev_ref
    out_ref[...] = _mlp_math(
        e0_ref[...], e1_ref[...], w1a_ref[...], w1b_ref[...], w2_ref[...],
        w3_ref[...], mavg_ref[...], b1_ref[...], b2_ref[...], b3_ref[...])


def _mlp_chunk(row_off, prev, e0c, e1c, weights):
    bs = min(_BS, e0c.shape[0])
    nblk = e0c.shape[0] // bs
    off = row_off // bs
    row_spec = pl.BlockSpec((bs, H), lambda i: (i, 0))
    w_spec = pl.BlockSpec((H, H), lambda i: (0, 0))
    b_spec = pl.BlockSpec((1, H), lambda i: (0, 0))
    out_spec = pl.BlockSpec((bs, H), lambda i: (i + off, 0))
    common = dict(
        grid=(nblk,),
        out_specs=out_spec,
        out_shape=jax.ShapeDtypeStruct((B, H), jnp.float32),
    )
    wspecs = [w_spec] * 5 + [b_spec] * 3
    if prev is None:
        return pl.pallas_call(
            _mlp_body,
            in_specs=[row_spec, row_spec] + wspecs,
            **common,
        )(e0c, e1c, *weights)
    return pl.pallas_call(
        _mlp_body_aliased,
        in_specs=[pl.BlockSpec(memory_space=pl.ANY), row_spec, row_spec]
                 + wspecs,
        input_output_aliases={0: 0},
        **common,
    )(prev, e0c, e1c, *weights)


def kernel(x, E0, E1,
           W1_0, b1_0, W2_0, b2_0, W3_0, b3_0,
           W1_1, b1_1, W2_1, b2_1, W3_1, b3_1):
    w1 = jnp.concatenate([W1_0.T, W1_1.T], axis=1)
    w1a, w1b = w1[:H], w1[H:]
    zblk = jnp.zeros((D, D), jnp.float32)
    w2bd = jnp.block([[W2_0.T, zblk], [zblk, W2_1.T]])
    w3bd = jnp.block([[W3_0.T, zblk], [zblk, W3_1.T]])
    ones = jnp.full((D, D), 1.0 / D, jnp.float32)
    mavg = jnp.block([[ones, zblk], [zblk, ones]])
    b1 = jnp.concatenate([b1_0, b1_1]).reshape(1, 2 * D)
    b2 = jnp.concatenate([b2_0, b2_1]).reshape(1, 2 * D)
    b3 = jnp.concatenate([b3_0, b3_1]).reshape(1, 2 * D)
    weights = (w1a, w1b, w2bd, w3bd, mavg, b1, b2, b3)

    x0 = x[:, 0].reshape(B // _CHUNK, _CHUNK)
    x1 = x[:, 1].reshape(B // _CHUNK, _CHUNK)
    pairs = []
    off = 0
    for nr in _CHUNKS:
        r0, r1 = off // _CHUNK, (off + nr) // _CHUNK
        pairs.append((off, _make_sc_gather(nr)(x0[r0:r1], x1[r0:r1], E0, E1)))
        off += nr
    out = None
    for off, (e0c, e1c) in pairs:
        out = _mlp_chunk(off, out, e0c, e1c, weights)
    return out
